# Initial kernel scaffold; baseline (speedup 1.0000x reference)
import jax
import jax.numpy as jnp
from jax.experimental import pallas as pl


def kernel(z_orig, z_augs):
    raise NotImplementedError("write your pallas kernel here")


# speedup vs baseline: 3.3033x; 3.3033x over previous
import functools

import jax
import jax.numpy as jnp
from jax.experimental import pallas as pl
from jax.experimental.pallas import tpu as pltpu

_TAU = 0.1
_EPS = 1e-5
_B = 8
_NB = 16
_C = 64


def _temporal_body(z_ref, zt_ref, c2_ref, cst_ref, sall_ref, sdiag_ref, *, T, RB):
    rx = pl.program_id(1)
    zb = z_ref[0]
    zbt = zt_ref[0]
    rows = z_ref[0, pl.ds(rx * RB, RB), :]
    gram = jax.lax.dot_general(rows, zbt, (((1,), (0,)), ((), ())),
                               preferred_element_type=jnp.float32)
    scaled = gram * (1.0 / _TAU)
    norms = jnp.sum(zbt * zbt, axis=0, keepdims=True)
    cst = jnp.max(norms) * (1.0 / _TAU)
    col = jax.lax.broadcasted_iota(jnp.int32, (RB, T), 1)
    row = rx * RB + jax.lax.broadcasted_iota(jnp.int32, (RB, T), 0)
    e = jnp.where(col > row, jnp.exp(scaled - cst), 0.0)
    denom = jnp.sum(e, axis=1, keepdims=True)
    wi = (T - 1) - (rx * RB + jax.lax.broadcasted_iota(jnp.int32, (RB, 1), 0))
    w = wi.astype(jnp.float32)
    c2 = jnp.sum(w * jnp.log(denom + _EPS))
    c2_ref[...] = jnp.full(c2_ref.shape, c2, jnp.float32)
    sz = jnp.sum(zb, axis=0, keepdims=True)
    sall = jnp.sum(sz * sz) * (1.0 / _TAU)
    sdiag = jnp.sum(norms) * (1.0 / _TAU)
    cst_ref[...] = jnp.full(cst_ref.shape, cst, jnp.float32)
    sall_ref[...] = jnp.full(sall_ref.shape, sall, jnp.float32)
    sdiag_ref[...] = jnp.full(sdiag_ref.shape, sdiag, jnp.float32)


def _temporal_level(z, zt, T, RB):
    nrb = T // RB
    c2, cst, sall, sdiag = pl.pallas_call(
        functools.partial(_temporal_body, T=T, RB=RB),
        grid=(_NB, nrb),
        in_specs=[
            pl.BlockSpec((1, T, _C), lambda b, r: (b, 0, 0)),
            pl.BlockSpec((1, _C, T), lambda b, r: (b, 0, 0)),
        ],
        out_specs=(
            pl.BlockSpec((1, 1, 1, 128), lambda b, r: (b, r, 0, 0)),
            pl.BlockSpec((1, 1, 128), lambda b, r: (b, 0, 0)),
            pl.BlockSpec((1, 1, 128), lambda b, r: (b, 0, 0)),
            pl.BlockSpec((1, 1, 128), lambda b, r: (b, 0, 0)),
        ),
        out_shape=(
            jax.ShapeDtypeStruct((_NB, nrb, 1, 128), jnp.float32),
            jax.ShapeDtypeStruct((_NB, 1, 128), jnp.float32),
            jax.ShapeDtypeStruct((_NB, 1, 128), jnp.float32),
            jax.ShapeDtypeStruct((_NB, 1, 128), jnp.float32),
        ),
        compiler_params=pltpu.CompilerParams(
            dimension_semantics=("parallel", "arbitrary"),
        ),
        name=f"temporal_T{T}",
    )(z, zt)
    c2s = jnp.sum(c2[:, :, 0, 0], axis=1)
    upper = (cst[:, 0, 0] * (T * (T - 1) / 2.0) + c2s
             - (sall[:, 0, 0] - sdiag[:, 0, 0]) * 0.5)
    return jnp.sum(upper) / (_B * T * (T - 1))


def _gram_lse_sum(z):
    gs = []
    for j in range(_NB):
        gs.append(jnp.sum(z * z[j:j + 1], axis=-1))
    g = jnp.stack(gs, axis=1)
    ii = jax.lax.broadcasted_iota(jnp.int32, (_NB, _NB, 1), 0)
    jj = jax.lax.broadcasted_iota(jnp.int32, (_NB, _NB, 1), 1)
    off = ii != jj
    m = jnp.max(jnp.where(off, g, jnp.float32(-1e30)), axis=1, keepdims=True)
    s = jnp.sum(jnp.where(off, jnp.exp(g - m), 0.0), axis=1, keepdims=True)
    lse = m + jnp.log(s)
    pos = jnp.sum(jnp.where(jj == ((ii + _B) % _NB), g, 0.0), axis=1,
                  keepdims=True)
    return jnp.sum(lse - pos)


def _instance_body_mid(z_ref, inst_ref, pool_ref, *, TB):
    z = z_ref[...]
    inst_ref[...] = jnp.full(inst_ref.shape, _gram_lse_sum(z), jnp.float32)
    zp = z.reshape(_NB, TB // 2, 2, _C)
    pool_ref[...] = jnp.max(zp, axis=2)


def _instance_body_last(z_ref, inst_ref, fin_ref, pool_ref, *, TB):
    z = z_ref[...]
    inst_ref[...] = jnp.full(inst_ref.shape, _gram_lse_sum(z), jnp.float32)
    zp = jnp.max(z.reshape(_NB, TB // 2, 2, _C), axis=2)
    pool_ref[...] = zp
    fin_ref[...] = jnp.full(fin_ref.shape, _gram_lse_sum(zp), jnp.float32)


def _instance_level(z, T, TB):
    ntb = T // TB
    last = T == 2
    out_specs = [
        pl.BlockSpec((1, 1, 128), lambda t: (t, 0, 0)),
        pl.BlockSpec((_NB, TB // 2, _C), lambda t: (0, t, 0)),
    ]
    out_shape = [
        jax.ShapeDtypeStruct((ntb, 1, 128), jnp.float32),
        jax.ShapeDtypeStruct((_NB, T // 2, _C), jnp.float32),
    ]
    if last:
        body = functools.partial(_instance_body_last, TB=TB)
        out_specs.insert(1, pl.BlockSpec((1, 1, 128), lambda t: (t, 0, 0)))
        out_shape.insert(1, jax.ShapeDtypeStruct((ntb, 1, 128), jnp.float32))
    else:
        body = functools.partial(_instance_body_mid, TB=TB)
    outs = pl.pallas_call(
        body,
        grid=(ntb,),
        in_specs=[pl.BlockSpec((_NB, TB, _C), lambda t: (0, t, 0))],
        out_specs=tuple(out_specs),
        out_shape=tuple(out_shape),
        compiler_params=pltpu.CompilerParams(
            dimension_semantics=("parallel",),
        ),
        name=f"instance_T{T}",
    )(z)
    if last:
        inst, fin, zp = outs
        return (jnp.sum(inst[:, 0, 0]) / (T * _NB), zp,
                jnp.sum(fin[:, 0, 0]) / _NB)
    inst, zp = outs
    return jnp.sum(inst[:, 0, 0]) / (T * _NB), zp, None


def kernel(z_orig, z_augs):
    z = jnp.concatenate([z_orig, z_augs], axis=0)
    T = z.shape[1]
    total = jnp.float32(0.0)
    fin = None
    d = 0
    while T > 1:
        zt = jnp.swapaxes(z, 1, 2)
        temp = _temporal_level(z, zt, T, min(T, 512))
        inst, z, fin = _instance_level(z, T, min(T, 512))
        total = total + 0.5 * (inst + temp)
        d += 1
        T //= 2
    total = total + 0.5 * fin
    d += 1
    return total / d


# speedup vs baseline: 3.9047x; 1.1821x over previous
import jax
import jax.numpy as jnp
from jax.experimental import pallas as pl
from jax.experimental.pallas import tpu as pltpu

_TAU = 0.1
_INV_TAU = 10.0
_EPS = 1e-5
_B = 8
_NB = 16
_C = 64
_T0 = 2048
_H = 1024

_OFFS = [0, 1024, 1536, 1792, 1920, 1984, 2016, 2032, 2040, 2044, 2046]
_SIZES = [1024, 512, 256, 128, 64, 32, 16, 8, 4, 2, 1]
_SLAB_OFF = 1920
_SLAB_BOUNDS = [(0, 64), (64, 96), (96, 112), (112, 120), (120, 124),
                (124, 126), (126, 127)]


def _pool_body(z_ref, zc_ref):
    cur = z_ref[...]
    for d in range(8):
        size = _SIZES[d]
        cur = jnp.max(cur.reshape(1, size, 2, _C), axis=2)
        zc_ref[:, _OFFS[d]:_OFFS[d] + size, :] = cur
    p9 = jnp.max(cur.reshape(1, 4, 2, _C), axis=2)
    p10 = jnp.max(p9.reshape(1, 2, 2, _C), axis=2)
    p11 = jnp.max(p10.reshape(1, 1, 2, _C), axis=2)
    tail = jnp.concatenate(
        [p9, p10, p11, jnp.zeros((1, 1, _C), jnp.float32)], axis=1)
    zc_ref[:, 2040:2048, :] = tail


def _pool(z):
    return pl.pallas_call(
        _pool_body,
        grid=(_NB,),
        in_specs=[pl.BlockSpec((1, _T0, _C), lambda b: (b, 0, 0))],
        out_specs=pl.BlockSpec((1, _T0, _C), lambda b: (b, 0, 0)),
        out_shape=jax.ShapeDtypeStruct((_NB, _T0, _C), jnp.float32),
        compiler_params=pltpu.CompilerParams(
            dimension_semantics=("parallel",),
        ),
        name="pool_levels",
    )(z)


def _lane():
    return jax.lax.broadcasted_iota(jnp.int32, (1, 1, 128), 2)


def _gram_t(rows, cols):
    return jax.lax.dot_general(rows, cols, (((1,), (1,)), ((), ())),
                               preferred_element_type=jnp.float32)


_K2 = _INV_TAU * 1.4426950408889634


def _l0_tile(z_ref, ry, cx, tri):
    rows = z_ref[0, ry * _H:(ry + 1) * _H, :]
    cols = z_ref[0, cx * _H:(cx + 1) * _H, :]
    gram = _gram_t(rows, cols)
    zb = z_ref[0]
    norms = jnp.sum(zb * zb, axis=1, keepdims=True)
    mx = jnp.max(norms)
    cst = mx * _INV_TAU
    ex = jnp.exp2(gram * _K2 - mx * _K2)
    if tri:
        y = jax.lax.broadcasted_iota(jnp.int32, (_H, _H), 0)
        x = jax.lax.broadcasted_iota(jnp.int32, (_H, _H), 1)
        ex = jnp.where(y > x, ex, 0.0)
    dpart = jnp.sum(ex, axis=0, keepdims=True)
    return dpart, cst, norms


def _l0_c2(dtot, cx):
    x = cx * _H + jax.lax.broadcasted_iota(jnp.int32, (1, _H), 1)
    w = (_T0 - 1 - x).astype(jnp.float32)
    return jnp.sum(w * jnp.log(dtot + _EPS))


def _temporal_all_body(z_ref, zc_ref, res_ref, dacc_ref):
    k = pl.program_id(1)
    lane = _lane()

    @pl.when(k == 0)
    def _t00():
        dpart, cst, norms = _l0_tile(z_ref, 0, 0, tri=True)
        dacc_ref[...] = dpart
        zb = z_ref[0]
        sz = jnp.sum(zb, axis=0, keepdims=True)
        sall = jnp.sum(sz * sz) * _INV_TAU
        sdiag = jnp.sum(norms) * _INV_TAU
        res_ref[...] = (jnp.where(lane == 16, cst, 0.0)
                        + jnp.where(lane == 32, sall, 0.0)
                        + jnp.where(lane == 48, sdiag, 0.0))

    @pl.when(k == 1)
    def _t10():
        dpart, _, _ = _l0_tile(z_ref, 1, 0, tri=False)
        c2 = _l0_c2(dacc_ref[...] + dpart, 0)
        res_ref[...] = res_ref[...] + jnp.where(lane == 0, c2, 0.0)

    @pl.when(k == 2)
    def _t11():
        dpart, _, _ = _l0_tile(z_ref, 1, 1, tri=True)
        c2 = _l0_c2(dpart, 1)
        res_ref[...] = res_ref[...] + jnp.where(lane == 0, c2, 0.0)

    @pl.when(k == 3)
    def _rest():
        pack = jnp.zeros((1, 1, 128), jnp.float32)
        for d in range(1, 5):
            off, T = _OFFS[d - 1], _SIZES[d - 1]
            zl = zc_ref[0, off:off + T, :]
            gram = _gram_t(zl, zl)
            norms = jnp.sum(zl * zl, axis=1, keepdims=True)
            mx = jnp.max(norms)
            cst = mx * _INV_TAU
            y = jax.lax.broadcasted_iota(jnp.int32, (T, T), 0)
            x = jax.lax.broadcasted_iota(jnp.int32, (T, T), 1)
            e = jnp.where(y > x, jnp.exp2(gram * _K2 - mx * _K2), 0.0)
            denom = jnp.sum(e, axis=0, keepdims=True)
            w = ((T - 1)
                 - jax.lax.broadcasted_iota(jnp.int32, (1, T), 1)
                 ).astype(jnp.float32)
            c2 = jnp.sum(w * jnp.log(denom + _EPS))
            sz = jnp.sum(zl, axis=0, keepdims=True)
            sall = jnp.sum(sz * sz) * _INV_TAU
            sdiag = jnp.sum(norms) * _INV_TAU
            pack = (pack + jnp.where(lane == d, c2, 0.0)
                    + jnp.where(lane == 16 + d, cst, 0.0)
                    + jnp.where(lane == 32 + d, sall, 0.0)
                    + jnp.where(lane == 48 + d, sdiag, 0.0))
        zs = zc_ref[0, _SLAB_OFF:_SLAB_OFF + 128, :]
        scaled = _gram_t(zs, zs) * _INV_TAU
        posx = jax.lax.broadcasted_iota(jnp.int32, (1, 128), 1)
        posY = jax.lax.broadcasted_iota(jnp.int32, (128, 128), 0)
        posX = jax.lax.broadcasted_iota(jnp.int32, (128, 128), 1)
        diagm = jnp.where(posY == posX, scaled, 0.0)
        norms = jnp.sum(diagm, axis=0, keepdims=True)
        lidY = jnp.zeros((128, 128), jnp.int32)
        lidX = jnp.zeros((128, 128), jnp.int32)
        for a, _e in _SLAB_BOUNDS[1:]:
            lidY = lidY + jnp.where(posY >= a, 1, 0)
            lidX = lidX + jnp.where(posX >= a, 1, 0)
        lidY = lidY + jnp.where(posY >= 127, 1, 0)
        lidX = lidX + jnp.where(posX >= 127, 1, 0)
        mask = (lidY == lidX) & (posY > posX)
        cstvec = jnp.zeros((1, 128), jnp.float32)
        csts = []
        for a, b2 in _SLAB_BOUNDS[:6]:
            inr = (posx >= a) & (posx < b2)
            ck = jnp.max(jnp.where(inr, norms, -1e30))
            csts.append(ck)
            cstvec = cstvec + jnp.where(inr, ck, 0.0)
        e = jnp.where(mask, jnp.exp(scaled - cstvec), 0.0)
        denom = jnp.sum(e, axis=0, keepdims=True)
        logd = jnp.log(denom + _EPS)
        endv = jnp.where(posx < 64, 64,
               jnp.where(posx < 96, 96,
               jnp.where(posx < 112, 112,
               jnp.where(posx < 120, 120,
               jnp.where(posx < 124, 124,
               jnp.where(posx < 126, 126, 127))))))
        w = jnp.maximum(endv - 1 - posx, 0).astype(jnp.float32)
        wl = w * logd
        rowi = jax.lax.broadcasted_iota(jnp.int32, (128, 1), 0)
        for i, (a, b2) in enumerate(_SLAB_BOUNDS[:6]):
            d = 5 + i
            inr = (posx >= a) & (posx < b2)
            c2k = jnp.sum(jnp.where(inr, wl, 0.0))
            rmask = (rowi >= a) & (rowi < b2)
            szk = jnp.sum(jnp.where(rmask, zs, 0.0), axis=0, keepdims=True)
            sallk = jnp.sum(szk * szk) * _INV_TAU
            sdk = jnp.sum(jnp.where(inr, norms, 0.0))
            pack = (pack + jnp.where(lane == d, c2k, 0.0)
                    + jnp.where(lane == 16 + d, csts[i], 0.0)
                    + jnp.where(lane == 32 + d, sallk, 0.0)
                    + jnp.where(lane == 48 + d, sdk, 0.0))
        res_ref[...] = res_ref[...] + pack


def _temporal_all(z, zc):
    return pl.pallas_call(
        _temporal_all_body,
        grid=(_NB, 4),
        in_specs=[
            pl.BlockSpec((1, _T0, _C), lambda b, r: (b, 0, 0)),
            pl.BlockSpec((1, _T0, _C), lambda b, r: (b, 0, 0)),
        ],
        out_specs=pl.BlockSpec((1, 1, 128), lambda b, r: (b, 0, 0)),
        out_shape=jax.ShapeDtypeStruct((_NB, 1, 128), jnp.float32),
        scratch_shapes=[pltpu.VMEM((1, _H), jnp.float32)],
        compiler_params=pltpu.CompilerParams(
            dimension_semantics=("parallel", "arbitrary"),
        ),
        name="temporal_all",
    )(z, zc)


def _inst_per_t(z, TC):
    gs = []
    for j in range(_NB):
        gs.append(jnp.sum(z * z[j:j + 1], axis=-1))
    g = jnp.stack(gs, axis=1)
    ii = jax.lax.broadcasted_iota(jnp.int32, (_NB, _NB, 1), 0)
    jj = jax.lax.broadcasted_iota(jnp.int32, (_NB, _NB, 1), 1)
    off = ii != jj
    m = jnp.max(jnp.where(off, g, jnp.float32(-1e30)), axis=1, keepdims=True)
    s = jnp.sum(jnp.where(off, jnp.exp(g - m), 0.0), axis=1, keepdims=True)
    lse = m + jnp.log(s)
    pos = jnp.sum(jnp.where(jj == ((ii + _B) % _NB), g, 0.0), axis=1,
                  keepdims=True)
    return jnp.sum(lse - pos, axis=0, keepdims=True)


def _instance_all_body(z_ref, zc_ref, out_ref):
    s = pl.program_id(0)
    lane = _lane()

    @pl.when(s < 4)
    def _l0():
        vt = _inst_per_t(z_ref[...], 512)
        out_ref[...] = jnp.where(lane == 0, jnp.sum(vt), 0.0)

    @pl.when(s == 4)
    def _l1():
        vt = _inst_per_t(zc_ref[...], 1024)
        out_ref[...] = jnp.where(lane == 1, jnp.sum(vt), 0.0)

    @pl.when(s == 5)
    def _rest():
        vt = _inst_per_t(zc_ref[...], 1024)
        ti = jax.lax.broadcasted_iota(jnp.int32, (1, 1, 1024), 2)
        pack = jnp.zeros((1, 1, 128), jnp.float32)
        for d in range(2, 12):
            a = _OFFS[d - 1] - 1024
            b2 = a + _SIZES[d - 1]
            vk = jnp.sum(jnp.where((ti >= a) & (ti < b2), vt, 0.0))
            pack = pack + jnp.where(lane == d, vk, 0.0)
        out_ref[...] = pack


def _instance_all(z, zc):
    return pl.pallas_call(
        _instance_all_body,
        grid=(6,),
        in_specs=[
            pl.BlockSpec((_NB, 512, _C), lambda s: (0, jnp.minimum(s, 3), 0)),
            pl.BlockSpec((_NB, 1024, _C),
                         lambda s: (0, jnp.maximum(s - 4, 0), 0)),
        ],
        out_specs=pl.BlockSpec((1, 1, 128), lambda s: (s, 0, 0)),
        out_shape=jax.ShapeDtypeStruct((6, 1, 128), jnp.float32),
        compiler_params=pltpu.CompilerParams(
            dimension_semantics=("arbitrary",),
        ),
        name="instance_all",
    )(z, zc)


def kernel(z_orig, z_augs):
    z = jnp.concatenate([z_orig, z_augs], axis=0)
    zc = _pool(z)
    res = _temporal_all(z, zc)
    inst = _instance_all(z, zc)

    total = jnp.float32(0.0)
    sizes = [_T0] + _SIZES
    for d in range(11):
        T = sizes[d]
        c2 = res[:, 0, d]
        cst = res[:, 0, 16 + d]
        sall = res[:, 0, 32 + d]
        sdiag = res[:, 0, 48 + d]
        upper = cst * (T * (T - 1) / 2.0) + c2 - (sall - sdiag) * 0.5
        total = total + 0.5 * jnp.sum(upper) / (_B * T * (T - 1))
    inst0 = (inst[0, 0, 0] + inst[1, 0, 0] + inst[2, 0, 0] + inst[3, 0, 0])
    total = total + 0.5 * inst0 / (_T0 * _NB)
    total = total + 0.5 * inst[4, 0, 1] / (1024 * _NB)
    for d in range(2, 12):
        total = total + 0.5 * inst[5, 0, d] / (_SIZES[d - 1] * _NB)
    return total / 12
